# baseline (speedup 1.0000x reference)
import jax
import jax.numpy as jnp
import numpy as np
from jax.experimental import pallas as pl
from jax.experimental.pallas import tpu as pltpu

_B, _C, _H, _W = 16, 1024, 32, 32
_HW = _H * _W
_T = _HW
_RC = 8

_KS0 = np.uint32(0)
_KS1 = np.uint32(42)
_KS2 = np.uint32(_KS0 ^ _KS1 ^ np.uint32(0x1BD11BDA))
_ROT = ((13, 15, 26, 6), (17, 29, 16, 24))


def _rounds(x0, x1, rs):
    for r in rs:
        x0 = x0 + x1
        x1 = (x1 << jnp.uint32(r)) | (x1 >> jnp.uint32(32 - r))
        x1 = x0 ^ x1
    return x0, x1


def _threefry_bits(cnt):
    x0 = jnp.zeros(cnt.shape, jnp.uint32) + jnp.uint32(_KS0)
    x1 = cnt + jnp.uint32(_KS1)
    x0, x1 = _rounds(x0, x1, _ROT[0])
    x0 = x0 + jnp.uint32(_KS1)
    x1 = x1 + jnp.uint32(_KS2 + np.uint32(1))
    x0, x1 = _rounds(x0, x1, _ROT[1])
    x0 = x0 + jnp.uint32(_KS2)
    x1 = x1 + jnp.uint32(_KS0 + np.uint32(2))
    x0, x1 = _rounds(x0, x1, _ROT[0])
    x0 = x0 + jnp.uint32(_KS0)
    x1 = x1 + jnp.uint32(_KS1 + np.uint32(3))
    x0, x1 = _rounds(x0, x1, _ROT[1])
    x0 = x0 + jnp.uint32(_KS1)
    x1 = x1 + jnp.uint32(_KS2 + np.uint32(4))
    x0, x1 = _rounds(x0, x1, _ROT[0])
    x0 = x0 + jnp.uint32(_KS2)
    x1 = x1 + jnp.uint32(_KS0 + np.uint32(5))
    return x0 ^ x1


def _onehot_argmax_kernel(x_ref, o_ref, bits_ref):
    b = pl.program_id(0).astype(jnp.uint32)
    base = b * jnp.uint32(_C * _HW)

    k = jax.lax.broadcasted_iota(jnp.uint32, (_RC, _T), 0)
    t = jax.lax.broadcasted_iota(jnp.uint32, (_RC, _T), 1)
    cnt0 = base + k * jnp.uint32(_HW) + t

    def body(i, carry):
        for half in range(2):
            c0 = i * 2 + half
            cnt = cnt0 + (c0 * (_RC * _HW)).astype(jnp.uint32)
            bits_ref[pl.ds(c0 * _RC, _RC), :] = _threefry_bits(cnt)
        return carry

    jax.lax.fori_loop(0, _C // (2 * _RC), body, 0)

    bits = bits_ref[...]
    fb = (bits >> jnp.uint32(9)) | jnp.uint32(0x3F800000)
    f = jax.lax.bitcast_convert_type(fb, jnp.float32) - jnp.float32(1.0)
    tiny = jnp.float32(np.finfo(np.float32).tiny)
    span = jnp.float32(np.float32(1.0) - np.finfo(np.float32).tiny)
    u = jnp.maximum(tiny, f * span + tiny)
    g = -jnp.log(-jnp.log(u))

    s = x_ref[0] + g
    idx = jnp.argmax(s, axis=0)
    iota = jax.lax.broadcasted_iota(jnp.int32, (_C, _T), 0)
    o_ref[0] = (iota == idx[None, :]).astype(jnp.float32)


def kernel(x):
    xr = x.reshape(_B, _C, _HW)
    out = pl.pallas_call(
        _onehot_argmax_kernel,
        grid=(_B,),
        in_specs=[
            pl.BlockSpec((1, _C, _T), lambda b: (b, 0, 0)),
        ],
        out_specs=pl.BlockSpec((1, _C, _T), lambda b: (b, 0, 0)),
        out_shape=jax.ShapeDtypeStruct((_B, _C, _HW), jnp.float32),
        scratch_shapes=[pltpu.VMEM((_C, _T), jnp.uint32)],
    )(xr)
    return out.reshape(_B, _C, _H, _W)
